# Initial kernel scaffold; baseline (speedup 1.0000x reference)
import functools

import jax
import jax.numpy as jnp
from jax import lax
from jax.experimental import pallas as pl
from jax.experimental.pallas import tpu as pltpu

N = 100000
E = 1600000
H = 16
NF = 100.0

EBLK = 12800
NBLK = 5000


def _silu(z):
    return z * jax.nn.sigmoid(z)


def _edge0_body(r_ref, c_ref, attr_ref, w1a_ref, w1b_ref, w1c_ref, b1_ref,
                w2_ref, b2_ref, aw_ref, ab_ref, ef_ref, aux_ref):
    r = r_ref[...]
    c = c_ref[...]
    hr = r[:, 0:16]
    hc = c[:, 0:16]
    xr = r[:, 16:19]
    xc = c[:, 16:19]
    cd = xr - xc
    radial = jnp.sum(cd * cd, axis=1, keepdims=True)
    norm = jnp.sqrt(radial + 1e-8)
    cdn = cd / norm
    attr = attr_ref[...]
    z1 = (jnp.dot(hr, w1a_ref[...], preferred_element_type=jnp.float32)
          + jnp.dot(hc, w1b_ref[...], preferred_element_type=jnp.float32)
          + radial * w1c_ref[0:1, :] + attr * w1c_ref[1:2, :] + b1_ref[...])
    m1 = _silu(z1)
    z2 = jnp.dot(m1, w2_ref[...], preferred_element_type=jnp.float32) + b2_ref[...]
    mij = _silu(z2)
    s = jnp.dot(mij, aw_ref[...], preferred_element_type=jnp.float32) + ab_ref[...]
    att = jax.nn.sigmoid(s)
    ef_ref[...] = mij * att
    zeros3 = jnp.zeros_like(cdn)
    aux_ref[...] = jnp.concatenate([radial, attr, cdn, zeros3], axis=1)


def _edge0(R, C, attr, w1, b1, w2, b2, aw, ab):
    nblk = E // EBLK
    w1a = w1[0:16]
    w1b = w1[16:32]
    w1c = w1[32:34]
    bspec = lambda bb, bw: pl.BlockSpec((bb, bw), lambda i: (i, 0))
    wspec = lambda a: pl.BlockSpec(a.shape, lambda i: (0,) * a.ndim)
    return pl.pallas_call(
        _edge0_body,
        grid=(nblk,),
        in_specs=[bspec(EBLK, 32), bspec(EBLK, 32), bspec(EBLK, 1),
                  wspec(w1a), wspec(w1b), wspec(w1c), wspec(b1.reshape(1, 16)),
                  wspec(w2), wspec(b2.reshape(1, 16)), wspec(aw),
                  wspec(ab.reshape(1, 1))],
        out_specs=[bspec(EBLK, 16), bspec(EBLK, 8)],
        out_shape=[jax.ShapeDtypeStruct((E, 16), jnp.float32),
                   jax.ShapeDtypeStruct((E, 8), jnp.float32)],
    )(R, C, attr, w1a, w1b, w1c, b1.reshape(1, 16), w2, b2.reshape(1, 16),
      aw, ab.reshape(1, 1))


def _edge1_body(r_ref, c_ref, aux_ref, w1a_ref, w1b_ref, w1c_ref, b1_ref,
                w2_ref, b2_ref, aw_ref, ab_ref, ef_ref):
    hr = r_ref[...]
    hc = c_ref[...]
    aux = aux_ref[...]
    radial = aux[:, 0:1]
    attr = aux[:, 1:2]
    z1 = (jnp.dot(hr, w1a_ref[...], preferred_element_type=jnp.float32)
          + jnp.dot(hc, w1b_ref[...], preferred_element_type=jnp.float32)
          + radial * w1c_ref[0:1, :] + attr * w1c_ref[1:2, :] + b1_ref[...])
    m1 = _silu(z1)
    z2 = jnp.dot(m1, w2_ref[...], preferred_element_type=jnp.float32) + b2_ref[...]
    mij = _silu(z2)
    s = jnp.dot(mij, aw_ref[...], preferred_element_type=jnp.float32) + ab_ref[...]
    att = jax.nn.sigmoid(s)
    ef_ref[...] = mij * att


def _edge1(R, C, aux, w1, b1, w2, b2, aw, ab):
    nblk = E // EBLK
    w1a = w1[0:16]
    w1b = w1[16:32]
    w1c = w1[32:34]
    bspec = lambda bb, bw: pl.BlockSpec((bb, bw), lambda i: (i, 0))
    wspec = lambda a: pl.BlockSpec(a.shape, lambda i: (0,) * a.ndim)
    return pl.pallas_call(
        _edge1_body,
        grid=(nblk,),
        in_specs=[bspec(EBLK, 16), bspec(EBLK, 16), bspec(EBLK, 8),
                  wspec(w1a), wspec(w1b), wspec(w1c), wspec(b1.reshape(1, 16)),
                  wspec(w2), wspec(b2.reshape(1, 16)), wspec(aw),
                  wspec(ab.reshape(1, 1))],
        out_specs=bspec(EBLK, 16),
        out_shape=jax.ShapeDtypeStruct((E, 16), jnp.float32),
    )(R, C, aux, w1a, w1b, w1c, b1.reshape(1, 16), w2, b2.reshape(1, 16),
      aw, ab.reshape(1, 1))


def _edgeq_body(r_ref, c_ref, aux_ref, w1a_ref, w1b_ref, w1c_ref, b1_ref,
                w2_ref, b2_ref, w3_ref, trans_ref):
    hr = r_ref[...]
    hc = c_ref[...]
    aux = aux_ref[...]
    radial = aux[:, 0:1]
    attr = aux[:, 1:2]
    cdn = aux[:, 2:5]
    z1 = (jnp.dot(hr, w1a_ref[...], preferred_element_type=jnp.float32)
          + jnp.dot(hc, w1b_ref[...], preferred_element_type=jnp.float32)
          + radial * w1c_ref[0:1, :] + attr * w1c_ref[1:2, :] + b1_ref[...])
    m1 = _silu(z1)
    z2 = jnp.dot(m1, w2_ref[...], preferred_element_type=jnp.float32) + b2_ref[...]
    m2 = _silu(z2)
    t = jnp.dot(m2, w3_ref[...], preferred_element_type=jnp.float32)
    trans = cdn * t
    pad = jnp.zeros((trans.shape[0], 13), jnp.float32)
    trans_ref[...] = jnp.concatenate([trans, pad], axis=1)


def _edgeq(R, C, aux, w1, b1, w2, b2, w3):
    nblk = E // EBLK
    w1a = w1[0:16]
    w1b = w1[16:32]
    w1c = w1[32:34]
    bspec = lambda bb, bw: pl.BlockSpec((bb, bw), lambda i: (i, 0))
    wspec = lambda a: pl.BlockSpec(a.shape, lambda i: (0,) * a.ndim)
    return pl.pallas_call(
        _edgeq_body,
        grid=(nblk,),
        in_specs=[bspec(EBLK, 16), bspec(EBLK, 16), bspec(EBLK, 8),
                  wspec(w1a), wspec(w1b), wspec(w1c), wspec(b1.reshape(1, 16)),
                  wspec(w2), wspec(b2.reshape(1, 16)), wspec(w3)],
        out_specs=bspec(EBLK, 16),
        out_shape=jax.ShapeDtypeStruct((E, 16), jnp.float32),
    )(R, C, aux, w1a, w1b, w1c, b1.reshape(1, 16), w2, b2.reshape(1, 16), w3)


def _node_body(h_ref, p0_ref, p1_ref, w1a_ref, w1b_ref, b1_ref, w2_ref,
               b2_ref, hn_ref):
    h = h_ref[...]
    agg = (p0_ref[...] + p1_ref[...]) * (1.0 / NF)
    z1 = (jnp.dot(h, w1a_ref[...], preferred_element_type=jnp.float32)
          + jnp.dot(agg, w1b_ref[...], preferred_element_type=jnp.float32)
          + b1_ref[...])
    m = _silu(z1)
    hn_ref[...] = h + jnp.dot(m, w2_ref[...], preferred_element_type=jnp.float32) + b2_ref[...]


def _node(h, p0, p1, nw1, nb1, nw2, nb2):
    nblk = N // NBLK
    w1a = nw1[0:16]
    w1b = nw1[16:32]
    bspec = lambda bb, bw: pl.BlockSpec((bb, bw), lambda i: (i, 0))
    wspec = lambda a: pl.BlockSpec(a.shape, lambda i: (0,) * a.ndim)
    return pl.pallas_call(
        _node_body,
        grid=(nblk,),
        in_specs=[bspec(NBLK, 16), bspec(NBLK, 16), bspec(NBLK, 16),
                  wspec(w1a), wspec(w1b), wspec(nb1.reshape(1, 16)),
                  wspec(nw2), wspec(nb2.reshape(1, 16))],
        out_specs=bspec(NBLK, 16),
        out_shape=jax.ShapeDtypeStruct((N, 16), jnp.float32),
    )(h, p0, p1, w1a, w1b, nb1.reshape(1, 16), nw2, nb2.reshape(1, 16))


def _coord_body(x_ref, p0_ref, p1_ref, xn_ref):
    agg = (p0_ref[...] + p1_ref[...]) * (1.0 / NF)
    xn_ref[...] = x_ref[...] + agg[:, 0:3]


def _coord(x, px0, px1):
    nblk = N // NBLK
    bspec = lambda bb, bw: pl.BlockSpec((bb, bw), lambda i: (i, 0))
    return pl.pallas_call(
        _coord_body,
        grid=(nblk,),
        in_specs=[bspec(NBLK, 3), bspec(NBLK, 16), bspec(NBLK, 16)],
        out_specs=bspec(NBLK, 3),
        out_shape=jax.ShapeDtypeStruct((N, 3), jnp.float32),
    )(x, px0, px1)


def _gather(table, row, col):
    return table[row], table[col]


def _scatter_partials(vals, row):
    half = E // 2
    p0 = jnp.zeros((N, 16), jnp.float32).at[row[:half]].add(vals[:half])
    p1 = jnp.zeros((N, 16), jnp.float32).at[row[half:]].add(vals[half:])
    return p0, p1


def kernel(h, x, edge_index, node_mask, edge_mask, edge_attr,
           g0_ew1, g0_eb1, g0_ew2, g0_eb2, g0_nw1, g0_nb1, g0_nw2, g0_nb2,
           g0_aw, g0_ab, g1_ew1, g1_eb1, g1_ew2, g1_eb2, g1_nw1, g1_nb1,
           g1_nw2, g1_nb2, g1_aw, g1_ab, eq_w1, eq_b1, eq_w2, eq_b2, eq_w3):
    row = edge_index[0]
    col = edge_index[1]

    T0 = jnp.concatenate([h, x, jnp.zeros((N, 13), jnp.float32)], axis=1)
    R0, C0 = _gather(T0, row, col)
    ef0, aux = _edge0(R0, C0, edge_attr, g0_ew1, g0_eb1, g0_ew2, g0_eb2,
                      g0_aw, g0_ab)
    p0a, p0b = _scatter_partials(ef0, row)
    h1 = _node(h, p0a, p0b, g0_nw1, g0_nb1, g0_nw2, g0_nb2)

    R1, C1 = _gather(h1, row, col)
    ef1 = _edge1(R1, C1, aux, g1_ew1, g1_eb1, g1_ew2, g1_eb2, g1_aw, g1_ab)
    p1a, p1b = _scatter_partials(ef1, row)
    h2 = _node(h1, p1a, p1b, g1_nw1, g1_nb1, g1_nw2, g1_nb2)

    R2, C2 = _gather(h2, row, col)
    trans = _edgeq(R2, C2, aux, eq_w1, eq_b1, eq_w2, eq_b2, eq_w3)
    pxa, pxb = _scatter_partials(trans, row)
    xn = _coord(x, pxa, pxb)

    return (h2, xn)


# speedup vs baseline: 1.1973x; 1.1973x over previous
import functools

import jax
import jax.numpy as jnp
from jax import lax
from jax.experimental import pallas as pl
from jax.experimental.pallas import tpu as pltpu

N = 100000
E = 1600000
H = 16
NF = 100.0

EBLK = 2560
NBLK = 4000


def _silu(z):
    return z * jax.nn.sigmoid(z)


def _edge0_body(r_ref, c_ref, attr_ref, w1a_ref, w1b_ref, w1c_ref, b1_ref,
                w2_ref, b2_ref, aw_ref, ab_ref, ef_ref, aux_ref):
    r = r_ref[...]
    c = c_ref[...]
    hr = r[:, 0:16]
    hc = c[:, 0:16]
    xr = r[:, 16:19]
    xc = c[:, 16:19]
    cd = xr - xc
    radial = jnp.sum(cd * cd, axis=1, keepdims=True)
    norm = jnp.sqrt(radial + 1e-8)
    cdn = cd / norm
    attr = attr_ref[...]
    z1 = (jnp.dot(hr, w1a_ref[...], preferred_element_type=jnp.float32)
          + jnp.dot(hc, w1b_ref[...], preferred_element_type=jnp.float32)
          + radial * w1c_ref[0:1, :] + attr * w1c_ref[1:2, :] + b1_ref[...])
    m1 = _silu(z1)
    z2 = jnp.dot(m1, w2_ref[...], preferred_element_type=jnp.float32) + b2_ref[...]
    mij = _silu(z2)
    s = jnp.dot(mij, aw_ref[...], preferred_element_type=jnp.float32) + ab_ref[...]
    att = jax.nn.sigmoid(s)
    ef_ref[...] = mij * att
    zeros3 = jnp.zeros_like(cdn)
    aux_ref[...] = jnp.concatenate([radial, attr, cdn, zeros3], axis=1)


def _edge0(R, C, attr, w1, b1, w2, b2, aw, ab):
    nblk = E // EBLK
    w1a = w1[0:16]
    w1b = w1[16:32]
    w1c = w1[32:34]
    bspec = lambda bb, bw: pl.BlockSpec((bb, bw), lambda i: (i, 0))
    wspec = lambda a: pl.BlockSpec(a.shape, lambda i: (0,) * a.ndim)
    return pl.pallas_call(
        _edge0_body,
        grid=(nblk,),
        in_specs=[bspec(EBLK, 32), bspec(EBLK, 32), bspec(EBLK, 1),
                  wspec(w1a), wspec(w1b), wspec(w1c), wspec(b1.reshape(1, 16)),
                  wspec(w2), wspec(b2.reshape(1, 16)), wspec(aw),
                  wspec(ab.reshape(1, 1))],
        out_specs=[bspec(EBLK, 16), bspec(EBLK, 8)],
        out_shape=[jax.ShapeDtypeStruct((E, 16), jnp.float32),
                   jax.ShapeDtypeStruct((E, 8), jnp.float32)],
    )(R, C, attr, w1a, w1b, w1c, b1.reshape(1, 16), w2, b2.reshape(1, 16),
      aw, ab.reshape(1, 1))


def _edge1_body(r_ref, c_ref, aux_ref, w1a_ref, w1b_ref, w1c_ref, b1_ref,
                w2_ref, b2_ref, aw_ref, ab_ref, ef_ref):
    hr = r_ref[...]
    hc = c_ref[...]
    aux = aux_ref[...]
    radial = aux[:, 0:1]
    attr = aux[:, 1:2]
    z1 = (jnp.dot(hr, w1a_ref[...], preferred_element_type=jnp.float32)
          + jnp.dot(hc, w1b_ref[...], preferred_element_type=jnp.float32)
          + radial * w1c_ref[0:1, :] + attr * w1c_ref[1:2, :] + b1_ref[...])
    m1 = _silu(z1)
    z2 = jnp.dot(m1, w2_ref[...], preferred_element_type=jnp.float32) + b2_ref[...]
    mij = _silu(z2)
    s = jnp.dot(mij, aw_ref[...], preferred_element_type=jnp.float32) + ab_ref[...]
    att = jax.nn.sigmoid(s)
    ef_ref[...] = mij * att


def _edge1(R, C, aux, w1, b1, w2, b2, aw, ab):
    nblk = E // EBLK
    w1a = w1[0:16]
    w1b = w1[16:32]
    w1c = w1[32:34]
    bspec = lambda bb, bw: pl.BlockSpec((bb, bw), lambda i: (i, 0))
    wspec = lambda a: pl.BlockSpec(a.shape, lambda i: (0,) * a.ndim)
    return pl.pallas_call(
        _edge1_body,
        grid=(nblk,),
        in_specs=[bspec(EBLK, 16), bspec(EBLK, 16), bspec(EBLK, 8),
                  wspec(w1a), wspec(w1b), wspec(w1c), wspec(b1.reshape(1, 16)),
                  wspec(w2), wspec(b2.reshape(1, 16)), wspec(aw),
                  wspec(ab.reshape(1, 1))],
        out_specs=bspec(EBLK, 16),
        out_shape=jax.ShapeDtypeStruct((E, 16), jnp.float32),
    )(R, C, aux, w1a, w1b, w1c, b1.reshape(1, 16), w2, b2.reshape(1, 16),
      aw, ab.reshape(1, 1))


def _edgeq_body(r_ref, c_ref, aux_ref, w1a_ref, w1b_ref, w1c_ref, b1_ref,
                w2_ref, b2_ref, w3_ref, trans_ref):
    hr = r_ref[...]
    hc = c_ref[...]
    aux = aux_ref[...]
    radial = aux[:, 0:1]
    attr = aux[:, 1:2]
    cdn = aux[:, 2:5]
    z1 = (jnp.dot(hr, w1a_ref[...], preferred_element_type=jnp.float32)
          + jnp.dot(hc, w1b_ref[...], preferred_element_type=jnp.float32)
          + radial * w1c_ref[0:1, :] + attr * w1c_ref[1:2, :] + b1_ref[...])
    m1 = _silu(z1)
    z2 = jnp.dot(m1, w2_ref[...], preferred_element_type=jnp.float32) + b2_ref[...]
    m2 = _silu(z2)
    t = jnp.dot(m2, w3_ref[...], preferred_element_type=jnp.float32)
    trans = cdn * t
    pad = jnp.zeros((trans.shape[0], 13), jnp.float32)
    trans_ref[...] = jnp.concatenate([trans, pad], axis=1)


def _edgeq(R, C, aux, w1, b1, w2, b2, w3):
    nblk = E // EBLK
    w1a = w1[0:16]
    w1b = w1[16:32]
    w1c = w1[32:34]
    bspec = lambda bb, bw: pl.BlockSpec((bb, bw), lambda i: (i, 0))
    wspec = lambda a: pl.BlockSpec(a.shape, lambda i: (0,) * a.ndim)
    return pl.pallas_call(
        _edgeq_body,
        grid=(nblk,),
        in_specs=[bspec(EBLK, 16), bspec(EBLK, 16), bspec(EBLK, 8),
                  wspec(w1a), wspec(w1b), wspec(w1c), wspec(b1.reshape(1, 16)),
                  wspec(w2), wspec(b2.reshape(1, 16)), wspec(w3)],
        out_specs=bspec(EBLK, 16),
        out_shape=jax.ShapeDtypeStruct((E, 16), jnp.float32),
    )(R, C, aux, w1a, w1b, w1c, b1.reshape(1, 16), w2, b2.reshape(1, 16), w3)


def _node_body(h_ref, p0_ref, p1_ref, w1a_ref, w1b_ref, b1_ref, w2_ref,
               b2_ref, hn_ref):
    h = h_ref[...]
    agg = (p0_ref[...] + p1_ref[...]) * (1.0 / NF)
    z1 = (jnp.dot(h, w1a_ref[...], preferred_element_type=jnp.float32)
          + jnp.dot(agg, w1b_ref[...], preferred_element_type=jnp.float32)
          + b1_ref[...])
    m = _silu(z1)
    hn_ref[...] = h + jnp.dot(m, w2_ref[...], preferred_element_type=jnp.float32) + b2_ref[...]


def _node(h, p0, p1, nw1, nb1, nw2, nb2):
    nblk = N // NBLK
    w1a = nw1[0:16]
    w1b = nw1[16:32]
    bspec = lambda bb, bw: pl.BlockSpec((bb, bw), lambda i: (i, 0))
    wspec = lambda a: pl.BlockSpec(a.shape, lambda i: (0,) * a.ndim)
    return pl.pallas_call(
        _node_body,
        grid=(nblk,),
        in_specs=[bspec(NBLK, 16), bspec(NBLK, 16), bspec(NBLK, 16),
                  wspec(w1a), wspec(w1b), wspec(nb1.reshape(1, 16)),
                  wspec(nw2), wspec(nb2.reshape(1, 16))],
        out_specs=bspec(NBLK, 16),
        out_shape=jax.ShapeDtypeStruct((N, 16), jnp.float32),
    )(h, p0, p1, w1a, w1b, nb1.reshape(1, 16), nw2, nb2.reshape(1, 16))


def _coord_body(x_ref, p0_ref, p1_ref, xn_ref):
    agg = (p0_ref[...] + p1_ref[...]) * (1.0 / NF)
    xn_ref[...] = x_ref[...] + agg[:, 0:3]


def _coord(x, px0, px1):
    nblk = N // NBLK
    bspec = lambda bb, bw: pl.BlockSpec((bb, bw), lambda i: (i, 0))
    return pl.pallas_call(
        _coord_body,
        grid=(nblk,),
        in_specs=[bspec(NBLK, 3), bspec(NBLK, 16), bspec(NBLK, 16)],
        out_specs=bspec(NBLK, 3),
        out_shape=jax.ShapeDtypeStruct((N, 3), jnp.float32),
    )(x, px0, px1)


def _gather(table, row, col):
    return table[row], table[col]


def _scatter_partials(vals, row):
    half = E // 2
    p0 = jnp.zeros((N, 16), jnp.float32).at[row[:half]].add(vals[:half])
    p1 = jnp.zeros((N, 16), jnp.float32).at[row[half:]].add(vals[half:])
    return p0, p1


def kernel(h, x, edge_index, node_mask, edge_mask, edge_attr,
           g0_ew1, g0_eb1, g0_ew2, g0_eb2, g0_nw1, g0_nb1, g0_nw2, g0_nb2,
           g0_aw, g0_ab, g1_ew1, g1_eb1, g1_ew2, g1_eb2, g1_nw1, g1_nb1,
           g1_nw2, g1_nb2, g1_aw, g1_ab, eq_w1, eq_b1, eq_w2, eq_b2, eq_w3):
    row = edge_index[0]
    col = edge_index[1]

    T0 = jnp.concatenate([h, x, jnp.zeros((N, 13), jnp.float32)], axis=1)
    R0, C0 = _gather(T0, row, col)
    ef0, aux = _edge0(R0, C0, edge_attr, g0_ew1, g0_eb1, g0_ew2, g0_eb2,
                      g0_aw, g0_ab)
    p0a, p0b = _scatter_partials(ef0, row)
    h1 = _node(h, p0a, p0b, g0_nw1, g0_nb1, g0_nw2, g0_nb2)

    R1, C1 = _gather(h1, row, col)
    ef1 = _edge1(R1, C1, aux, g1_ew1, g1_eb1, g1_ew2, g1_eb2, g1_aw, g1_ab)
    p1a, p1b = _scatter_partials(ef1, row)
    h2 = _node(h1, p1a, p1b, g1_nw1, g1_nb1, g1_nw2, g1_nb2)

    R2, C2 = _gather(h2, row, col)
    trans = _edgeq(R2, C2, aux, eq_w1, eq_b1, eq_w2, eq_b2, eq_w3)
    pxa, pxb = _scatter_partials(trans, row)
    xn = _coord(x, pxa, pxb)

    return (h2, xn)


# speedup vs baseline: 3.9286x; 3.2814x over previous
import functools

import jax
import jax.numpy as jnp
from jax import lax
from jax.experimental import pallas as pl
from jax.experimental.pallas import tpu as pltpu
from jax.experimental.pallas import tpu_sc as plsc

NC = 2
NS = 16

N = 100000
E = 1600000
H = 16
NF = 100.0

EBLK = 2560
NBLK = 4000


def _silu(z):
    return z * jax.nn.sigmoid(z)


def _edge0_body(r_ref, c_ref, attr_ref, w1a_ref, w1b_ref, w1c_ref, b1_ref,
                w2_ref, b2_ref, aw_ref, ab_ref, ef_ref, aux_ref):
    r = r_ref[...]
    c = c_ref[...]
    hr = r[:, 0:16]
    hc = c[:, 0:16]
    xr = r[:, 16:19]
    xc = c[:, 16:19]
    cd = xr - xc
    radial = jnp.sum(cd * cd, axis=1, keepdims=True)
    norm = jnp.sqrt(radial + 1e-8)
    cdn = cd / norm
    attr = attr_ref[...]
    z1 = (jnp.dot(hr, w1a_ref[...], preferred_element_type=jnp.float32)
          + jnp.dot(hc, w1b_ref[...], preferred_element_type=jnp.float32)
          + radial * w1c_ref[0:1, :] + attr * w1c_ref[1:2, :] + b1_ref[...])
    m1 = _silu(z1)
    z2 = jnp.dot(m1, w2_ref[...], preferred_element_type=jnp.float32) + b2_ref[...]
    mij = _silu(z2)
    s = jnp.dot(mij, aw_ref[...], preferred_element_type=jnp.float32) + ab_ref[...]
    att = jax.nn.sigmoid(s)
    ef_ref[...] = mij * att
    zeros3 = jnp.zeros_like(cdn)
    aux_ref[...] = jnp.concatenate([radial, attr, cdn, zeros3], axis=1)


def _edge0(R, C, attr, w1, b1, w2, b2, aw, ab):
    nblk = E // EBLK
    w1a = w1[0:16]
    w1b = w1[16:32]
    w1c = w1[32:34]
    bspec = lambda bb, bw: pl.BlockSpec((bb, bw), lambda i: (i, 0))
    wspec = lambda a: pl.BlockSpec(a.shape, lambda i: (0,) * a.ndim)
    return pl.pallas_call(
        _edge0_body,
        grid=(nblk,),
        in_specs=[bspec(EBLK, 32), bspec(EBLK, 32), bspec(EBLK, 1),
                  wspec(w1a), wspec(w1b), wspec(w1c), wspec(b1.reshape(1, 16)),
                  wspec(w2), wspec(b2.reshape(1, 16)), wspec(aw),
                  wspec(ab.reshape(1, 1))],
        out_specs=[bspec(EBLK, 16), bspec(EBLK, 8)],
        out_shape=[jax.ShapeDtypeStruct((E, 16), jnp.float32),
                   jax.ShapeDtypeStruct((E, 8), jnp.float32)],
    )(R, C, attr, w1a, w1b, w1c, b1.reshape(1, 16), w2, b2.reshape(1, 16),
      aw, ab.reshape(1, 1))


def _edge1_body(r_ref, c_ref, aux_ref, w1a_ref, w1b_ref, w1c_ref, b1_ref,
                w2_ref, b2_ref, aw_ref, ab_ref, ef_ref):
    hr = r_ref[...]
    hc = c_ref[...]
    aux = aux_ref[...]
    radial = aux[:, 0:1]
    attr = aux[:, 1:2]
    z1 = (jnp.dot(hr, w1a_ref[...], preferred_element_type=jnp.float32)
          + jnp.dot(hc, w1b_ref[...], preferred_element_type=jnp.float32)
          + radial * w1c_ref[0:1, :] + attr * w1c_ref[1:2, :] + b1_ref[...])
    m1 = _silu(z1)
    z2 = jnp.dot(m1, w2_ref[...], preferred_element_type=jnp.float32) + b2_ref[...]
    mij = _silu(z2)
    s = jnp.dot(mij, aw_ref[...], preferred_element_type=jnp.float32) + ab_ref[...]
    att = jax.nn.sigmoid(s)
    ef_ref[...] = mij * att


def _edge1(R, C, aux, w1, b1, w2, b2, aw, ab):
    nblk = E // EBLK
    w1a = w1[0:16]
    w1b = w1[16:32]
    w1c = w1[32:34]
    bspec = lambda bb, bw: pl.BlockSpec((bb, bw), lambda i: (i, 0))
    wspec = lambda a: pl.BlockSpec(a.shape, lambda i: (0,) * a.ndim)
    return pl.pallas_call(
        _edge1_body,
        grid=(nblk,),
        in_specs=[bspec(EBLK, 16), bspec(EBLK, 16), bspec(EBLK, 8),
                  wspec(w1a), wspec(w1b), wspec(w1c), wspec(b1.reshape(1, 16)),
                  wspec(w2), wspec(b2.reshape(1, 16)), wspec(aw),
                  wspec(ab.reshape(1, 1))],
        out_specs=bspec(EBLK, 16),
        out_shape=jax.ShapeDtypeStruct((E, 16), jnp.float32),
    )(R, C, aux, w1a, w1b, w1c, b1.reshape(1, 16), w2, b2.reshape(1, 16),
      aw, ab.reshape(1, 1))


def _edgeq_body(r_ref, c_ref, aux_ref, w1a_ref, w1b_ref, w1c_ref, b1_ref,
                w2_ref, b2_ref, w3_ref, trans_ref):
    hr = r_ref[...]
    hc = c_ref[...]
    aux = aux_ref[...]
    radial = aux[:, 0:1]
    attr = aux[:, 1:2]
    cdn = aux[:, 2:5]
    z1 = (jnp.dot(hr, w1a_ref[...], preferred_element_type=jnp.float32)
          + jnp.dot(hc, w1b_ref[...], preferred_element_type=jnp.float32)
          + radial * w1c_ref[0:1, :] + attr * w1c_ref[1:2, :] + b1_ref[...])
    m1 = _silu(z1)
    z2 = jnp.dot(m1, w2_ref[...], preferred_element_type=jnp.float32) + b2_ref[...]
    m2 = _silu(z2)
    t = jnp.dot(m2, w3_ref[...], preferred_element_type=jnp.float32)
    trans = cdn * t
    pad = jnp.zeros((trans.shape[0], 13), jnp.float32)
    trans_ref[...] = jnp.concatenate([trans, pad], axis=1)


def _edgeq(R, C, aux, w1, b1, w2, b2, w3):
    nblk = E // EBLK
    w1a = w1[0:16]
    w1b = w1[16:32]
    w1c = w1[32:34]
    bspec = lambda bb, bw: pl.BlockSpec((bb, bw), lambda i: (i, 0))
    wspec = lambda a: pl.BlockSpec(a.shape, lambda i: (0,) * a.ndim)
    return pl.pallas_call(
        _edgeq_body,
        grid=(nblk,),
        in_specs=[bspec(EBLK, 16), bspec(EBLK, 16), bspec(EBLK, 8),
                  wspec(w1a), wspec(w1b), wspec(w1c), wspec(b1.reshape(1, 16)),
                  wspec(w2), wspec(b2.reshape(1, 16)), wspec(w3)],
        out_specs=bspec(EBLK, 16),
        out_shape=jax.ShapeDtypeStruct((E, 16), jnp.float32),
    )(R, C, aux, w1a, w1b, w1c, b1.reshape(1, 16), w2, b2.reshape(1, 16), w3)


def _node_body(h_ref, p0_ref, p1_ref, w1a_ref, w1b_ref, b1_ref, w2_ref,
               b2_ref, hn_ref):
    h = h_ref[...]
    agg = (p0_ref[...] + p1_ref[...]) * (1.0 / NF)
    z1 = (jnp.dot(h, w1a_ref[...], preferred_element_type=jnp.float32)
          + jnp.dot(agg, w1b_ref[...], preferred_element_type=jnp.float32)
          + b1_ref[...])
    m = _silu(z1)
    hn_ref[...] = h + jnp.dot(m, w2_ref[...], preferred_element_type=jnp.float32) + b2_ref[...]


def _node(h, p0, p1, nw1, nb1, nw2, nb2):
    nblk = N // NBLK
    w1a = nw1[0:16]
    w1b = nw1[16:32]
    bspec = lambda bb, bw: pl.BlockSpec((bb, bw), lambda i: (i, 0))
    wspec = lambda a: pl.BlockSpec(a.shape, lambda i: (0,) * a.ndim)
    return pl.pallas_call(
        _node_body,
        grid=(nblk,),
        in_specs=[bspec(NBLK, 16), bspec(NBLK, 16), bspec(NBLK, 16),
                  wspec(w1a), wspec(w1b), wspec(nb1.reshape(1, 16)),
                  wspec(nw2), wspec(nb2.reshape(1, 16))],
        out_specs=bspec(NBLK, 16),
        out_shape=jax.ShapeDtypeStruct((N, 16), jnp.float32),
    )(h, p0, p1, w1a, w1b, nb1.reshape(1, 16), nw2, nb2.reshape(1, 16))


def _coord_body(x_ref, p0_ref, p1_ref, xn_ref):
    agg = (p0_ref[...] + p1_ref[...]) * (1.0 / NF)
    xn_ref[...] = x_ref[...] + agg[:, 0:3]


def _coord(x, px0, px1):
    nblk = N // NBLK
    bspec = lambda bb, bw: pl.BlockSpec((bb, bw), lambda i: (i, 0))
    return pl.pallas_call(
        _coord_body,
        grid=(nblk,),
        in_specs=[bspec(NBLK, 3), bspec(NBLK, 16), bspec(NBLK, 16)],
        out_specs=bspec(NBLK, 3),
        out_shape=jax.ShapeDtypeStruct((N, 3), jnp.float32),
    )(x, px0, px1)


GCH = 1000


def _sc_gather2(table, row, col):
    D = table.shape[1]
    per_w = E // (NC * NS)
    nch = per_w // GCH

    mesh = plsc.VectorSubcoreMesh(core_axis_name="c", subcore_axis_name="s")

    @functools.partial(
        pl.kernel, mesh=mesh,
        compiler_params=pltpu.CompilerParams(use_tc_tiling_on_sc=False),
        out_type=[jax.ShapeDtypeStruct((E, D), jnp.float32),
                  jax.ShapeDtypeStruct((E, D), jnp.float32)],
        scratch_types=[pltpu.VMEM((GCH,), jnp.int32),
                       pltpu.VMEM((GCH, D), jnp.float32),
                       pltpu.VMEM((GCH,), jnp.int32),
                       pltpu.VMEM((GCH, D), jnp.float32),
                       pltpu.SemaphoreType.DMA,
                       pltpu.SemaphoreType.DMA],
    )
    def k(table_hbm, row_hbm, col_hbm, outr_hbm, outc_hbm,
          ridx_v, rrows_v, cidx_v, crows_v, sem1, sem2):
        c = lax.axis_index("c")
        s = lax.axis_index("s")
        base = (c * NS + s) * per_w

        def body(i, carry):
            off = base + i * GCH
            pltpu.sync_copy(row_hbm.at[pl.ds(off, GCH)], ridx_v)
            pltpu.sync_copy(col_hbm.at[pl.ds(off, GCH)], cidx_v)
            cp1 = pltpu.async_copy(table_hbm.at[ridx_v], rrows_v, sem1)
            cp2 = pltpu.async_copy(table_hbm.at[cidx_v], crows_v, sem2)
            cp1.wait()
            cp2.wait()
            pltpu.sync_copy(rrows_v, outr_hbm.at[pl.ds(off, GCH)])
            pltpu.sync_copy(crows_v, outc_hbm.at[pl.ds(off, GCH)])
            return carry

        lax.fori_loop(0, nch, body, 0)

    return k(table, row, col)


SCH = 1000
ZROWS = 625


def _sc_scatter_partials(vals, row):
    per_core = E // NC
    per_w = per_core // NS
    nch = per_w // SCH
    rows_per_tile = N // NS

    mesh = plsc.VectorSubcoreMesh(core_axis_name="c", subcore_axis_name="s")

    @functools.partial(
        pl.kernel, mesh=mesh,
        compiler_params=pltpu.CompilerParams(use_tc_tiling_on_sc=False),
        out_type=jax.ShapeDtypeStruct((NC, N, 16), jnp.float32),
        scratch_types=[pltpu.VMEM((SCH,), jnp.int32),
                       pltpu.VMEM((SCH, 16), jnp.float32),
                       pltpu.VMEM((ZROWS, 16), jnp.float32),
                       pltpu.VMEM_SHARED((N, 16), jnp.float32)],
    )
    def k(vals_hbm, row_hbm, out_hbm, idx_v, val_v, zb_v, table_sh):
        c = lax.axis_index("c")
        s = lax.axis_index("s")

        def zb(i, carry):
            zb_v[i, :] = jnp.zeros((16,), jnp.float32)
            return carry

        lax.fori_loop(0, ZROWS, zb, 0)

        tbase = s * rows_per_tile

        def zt(j, carry):
            pltpu.sync_copy(zb_v, table_sh.at[pl.ds(tbase + j * ZROWS, ZROWS)])
            return carry

        lax.fori_loop(0, rows_per_tile // ZROWS, zt, 0)
        plsc.subcore_barrier()

        base = c * per_core + s * per_w

        def body(i, carry):
            off = base + i * SCH
            pltpu.sync_copy(row_hbm.at[pl.ds(off, SCH)], idx_v)
            pltpu.sync_copy(vals_hbm.at[pl.ds(off, SCH)], val_v)
            pltpu.sync_copy(val_v, table_sh.at[idx_v], add=True)
            return carry

        lax.fori_loop(0, nch, body, 0)
        plsc.subcore_barrier()

        pltpu.sync_copy(table_sh.at[pl.ds(tbase, rows_per_tile)],
                        out_hbm.at[c, pl.ds(tbase, rows_per_tile)])

    return k(vals, row)


def _gather(table, row, col):
    return _sc_gather2(table, row, col)


def _scatter_partials(vals, row):
    p = _sc_scatter_partials(vals, row)
    return p[0], p[1]


def kernel(h, x, edge_index, node_mask, edge_mask, edge_attr,
           g0_ew1, g0_eb1, g0_ew2, g0_eb2, g0_nw1, g0_nb1, g0_nw2, g0_nb2,
           g0_aw, g0_ab, g1_ew1, g1_eb1, g1_ew2, g1_eb2, g1_nw1, g1_nb1,
           g1_nw2, g1_nb2, g1_aw, g1_ab, eq_w1, eq_b1, eq_w2, eq_b2, eq_w3):
    row = edge_index[0]
    col = edge_index[1]

    T0 = jnp.concatenate([h, x, jnp.zeros((N, 13), jnp.float32)], axis=1)
    R0, C0 = _gather(T0, row, col)
    ef0, aux = _edge0(R0, C0, edge_attr, g0_ew1, g0_eb1, g0_ew2, g0_eb2,
                      g0_aw, g0_ab)
    p0a, p0b = _scatter_partials(ef0, row)
    h1 = _node(h, p0a, p0b, g0_nw1, g0_nb1, g0_nw2, g0_nb2)

    R1, C1 = _gather(h1, row, col)
    ef1 = _edge1(R1, C1, aux, g1_ew1, g1_eb1, g1_ew2, g1_eb2, g1_aw, g1_ab)
    p1a, p1b = _scatter_partials(ef1, row)
    h2 = _node(h1, p1a, p1b, g1_nw1, g1_nb1, g1_nw2, g1_nb2)

    R2, C2 = _gather(h2, row, col)
    trans = _edgeq(R2, C2, aux, eq_w1, eq_b1, eq_w2, eq_b2, eq_w3)
    pxa, pxb = _scatter_partials(trans, row)
    xn = _coord(x, pxa, pxb)

    return (h2, xn)


# speedup vs baseline: 12.2668x; 3.1224x over previous
import functools

import jax
import jax.numpy as jnp
from jax import lax
from jax.experimental import pallas as pl
from jax.experimental.pallas import tpu as pltpu
from jax.experimental.pallas import tpu_sc as plsc

NC = 2
NS = 16

N = 100000
E = 1600000
NF = 100.0

EBLK = 2560
EB8 = EBLK // 8
CBLK = 625
NBLK = 4000

GCH = 1000
GRP = 5


def _silu(z):
    return z * jax.nn.sigmoid(z)


def _kron8(w):
    return jnp.kron(jnp.eye(8, dtype=jnp.float32), w)


def _sc_gather(table, idx):
    D = table.shape[1]
    M = idx.shape[0]
    per_w = M // (NC * NS)
    nsup = per_w // (GCH * GRP)

    mesh = plsc.VectorSubcoreMesh(core_axis_name="c", subcore_axis_name="s")

    @functools.partial(
        pl.kernel, mesh=mesh,
        compiler_params=pltpu.CompilerParams(use_tc_tiling_on_sc=False),
        out_type=jax.ShapeDtypeStruct((M, D), jnp.float32),
        scratch_types=[pltpu.VMEM((GRP, GCH), jnp.int32),
                       pltpu.VMEM((GRP, GCH, D), jnp.float32),
                       pltpu.SemaphoreType.DMA,
                       pltpu.SemaphoreType.DMA,
                       pltpu.SemaphoreType.DMA],
    )
    def k(table_hbm, idx_hbm, out_hbm, idx_v, rows_v, sem_i, sem_g, sem_s):
        c = lax.axis_index("c")
        s = lax.axis_index("s")
        base = (c * NS + s) * per_w

        def sup(j, carry):
            off0 = base + j * (GCH * GRP)
            cps = [pltpu.async_copy(idx_hbm.at[pl.ds(off0 + b * GCH, GCH)],
                                    idx_v.at[b], sem_i) for b in range(GRP)]
            for cp in cps:
                cp.wait()
            gps = [pltpu.async_copy(table_hbm.at[idx_v.at[b]],
                                    rows_v.at[b], sem_g) for b in range(GRP)]
            for gp in gps:
                gp.wait()
            sps = [pltpu.async_copy(rows_v.at[b],
                                    out_hbm.at[pl.ds(off0 + b * GCH, GCH)],
                                    sem_s) for b in range(GRP)]
            for sp in sps:
                sp.wait()
            return carry

        lax.fori_loop(0, nsup, sup, 0)

    return k(table, idx)


SCH = 200
SGR = 5


def _sc_scatter_partials(vals, row):
    per_core = E // NC
    per_w = per_core // NS
    nsup = per_w // (SCH * SGR)
    rows_per_tile = N // NS

    mesh = plsc.VectorSubcoreMesh(core_axis_name="c", subcore_axis_name="s")

    @functools.partial(
        pl.kernel, mesh=mesh,
        compiler_params=pltpu.CompilerParams(use_tc_tiling_on_sc=False),
        out_type=jax.ShapeDtypeStruct((NC, N, 16), jnp.float32),
        scratch_types=[pltpu.VMEM((SGR, SCH), jnp.int32),
                       pltpu.VMEM((SGR, SCH, 16), jnp.float32),
                       pltpu.VMEM_SHARED((N, 16), jnp.float32),
                       pltpu.SemaphoreType.DMA],
    )
    def k(vals_hbm, row_hbm, zeros_hbm, out_hbm, idx_v, val_v, table_sh, sem):
        c = lax.axis_index("c")
        s = lax.axis_index("s")
        tbase = s * rows_per_tile

        pltpu.sync_copy(zeros_hbm.at[pl.ds(tbase, rows_per_tile)],
                        table_sh.at[pl.ds(tbase, rows_per_tile)])
        plsc.subcore_barrier()

        base = c * per_core + s * per_w

        def sup(j, carry):
            off0 = base + j * (SCH * SGR)
            cps = [pltpu.async_copy(row_hbm.at[pl.ds(off0 + b * SCH, SCH)],
                                    idx_v.at[b], sem) for b in range(SGR)]
            cps += [pltpu.async_copy(vals_hbm.at[pl.ds(off0 + b * SCH, SCH)],
                                     val_v.at[b], sem) for b in range(SGR)]
            for cp in cps:
                cp.wait()
            for b in range(SGR):
                pltpu.sync_copy(val_v.at[b], table_sh.at[idx_v.at[b]],
                                add=True)
            return carry

        lax.fori_loop(0, nsup, sup, 0)
        plsc.subcore_barrier()

        pltpu.sync_copy(table_sh.at[pl.ds(tbase, rows_per_tile)],
                        out_hbm.at[c, pl.ds(tbase, rows_per_tile)])

    return k(vals, row, jnp.zeros((N, 16), jnp.float32))


def _edge0p_body(hr_ref, hc_ref, xr_ref, xc_ref, attr_ref, w1a_ref, w1b_ref,
                 sq_ref, qx_ref, shr_ref, ex_ref, w1c0_ref, w1c1_ref, b1_ref,
                 w2_ref, b2_ref, awbd_ref, ab_ref, ef_ref, aux_ref):
    f32 = jnp.float32
    cd = xr_ref[...] - xc_ref[...]
    sq = cd * cd
    radial_rep = jnp.dot(sq, sq_ref[...], preferred_element_type=f32)
    radial_x = jnp.dot(sq, qx_ref[...], preferred_element_type=f32)
    inv = 1.0 / jnp.sqrt(radial_x + 1e-8)
    cdn = cd * inv
    cdn_aux = jnp.dot(cdn, shr_ref[...], preferred_element_type=f32)
    attr_rep = jnp.dot(attr_ref[...], ex_ref[...], preferred_element_type=f32)
    lane = lax.broadcasted_iota(jnp.int32, radial_rep.shape, 1) % 16
    aux_ref[...] = (jnp.where(lane == 0, radial_rep, 0.0)
                    + jnp.where(lane == 1, attr_rep, 0.0) + cdn_aux)
    z1 = (jnp.dot(hr_ref[...], w1a_ref[...], preferred_element_type=f32)
          + jnp.dot(hc_ref[...], w1b_ref[...], preferred_element_type=f32)
          + radial_rep * w1c0_ref[...] + attr_rep * w1c1_ref[...]
          + b1_ref[...])
    m1 = _silu(z1)
    z2 = jnp.dot(m1, w2_ref[...], preferred_element_type=f32) + b2_ref[...]
    mij = _silu(z2)
    att = jax.nn.sigmoid(jnp.dot(mij, awbd_ref[...], preferred_element_type=f32)
                         + ab_ref[...])
    ef_ref[...] = mij * att


def _edge0p(gh, gx, attrp, w1, b1, w2, b2, aw, ab):
    w1a_bd = _kron8(w1[0:16])
    w1b_bd = _kron8(w1[16:32])
    q = jnp.zeros((16, 16), jnp.float32).at[0:3, :].set(1.0)
    sq_bd = _kron8(q)
    qx = jnp.zeros((16, 16), jnp.float32).at[0:3, 0:3].set(
        jnp.ones((3, 3), jnp.float32))
    qx_bd = _kron8(qx)
    shr = jnp.zeros((16, 16), jnp.float32).at[jnp.arange(3), jnp.arange(2, 5)].set(1.0)
    shr_bd = _kron8(shr)
    ex = jnp.zeros((8, 128), jnp.float32)
    ex = ex.at[jnp.repeat(jnp.arange(8), 16),
               jnp.arange(128)].set(1.0)
    w1c0 = jnp.tile(w1[32], 8).reshape(1, 128)
    w1c1 = jnp.tile(w1[33], 8).reshape(1, 128)
    b1r = jnp.tile(b1, 8).reshape(1, 128)
    b2r = jnp.tile(b2, 8).reshape(1, 128)
    aw_bd = _kron8(aw @ jnp.ones((1, 16), jnp.float32))
    abr = jnp.full((1, 128), ab[0], jnp.float32)
    w2_bd = _kron8(w2)

    bspec = lambda nb, off: pl.BlockSpec((nb, 128), lambda i, o=off: (i + o, 0))
    aspec = pl.BlockSpec((EBLK // 8, 8), lambda i: (i, 0))
    wspec = lambda a: pl.BlockSpec(a.shape, lambda i: (0,) * a.ndim)
    outs = pl.pallas_call(
        _edge0p_body,
        grid=(E // EBLK,),
        in_specs=[bspec(EB8, 0), bspec(EB8, CBLK), bspec(EB8, 0),
                  bspec(EB8, CBLK), aspec,
                  wspec(w1a_bd), wspec(w1b_bd), wspec(sq_bd), wspec(qx_bd),
                  wspec(shr_bd), wspec(ex), wspec(w1c0), wspec(w1c1),
                  wspec(b1r), wspec(w2_bd), wspec(b2r), wspec(aw_bd),
                  wspec(abr)],
        out_specs=[bspec(EB8, 0), bspec(EB8, 0)],
        out_shape=[jax.ShapeDtypeStruct((E // 8, 128), jnp.float32),
                   jax.ShapeDtypeStruct((E // 8, 128), jnp.float32)],
    )(gh, gh, gx, gx, attrp, w1a_bd, w1b_bd, sq_bd, qx_bd, shr_bd, ex,
      w1c0, w1c1, b1r, w2_bd, b2r, aw_bd, abr)
    return outs


def _edge1p_body(hr_ref, hc_ref, aux_ref, w1a_ref, w1b_ref, sx_ref, b1_ref,
                 w2_ref, b2_ref, awbd_ref, ab_ref, ef_ref):
    f32 = jnp.float32
    z1 = (jnp.dot(hr_ref[...], w1a_ref[...], preferred_element_type=f32)
          + jnp.dot(hc_ref[...], w1b_ref[...], preferred_element_type=f32)
          + jnp.dot(aux_ref[...], sx_ref[...], preferred_element_type=f32)
          + b1_ref[...])
    m1 = _silu(z1)
    z2 = jnp.dot(m1, w2_ref[...], preferred_element_type=f32) + b2_ref[...]
    mij = _silu(z2)
    att = jax.nn.sigmoid(jnp.dot(mij, awbd_ref[...], preferred_element_type=f32)
                         + ab_ref[...])
    ef_ref[...] = mij * att


def _edge1p(gh, auxp, w1, b1, w2, b2, aw, ab):
    w1a_bd = _kron8(w1[0:16])
    w1b_bd = _kron8(w1[16:32])
    p = jnp.zeros((16, 16), jnp.float32).at[0, :].set(w1[32]).at[1, :].set(w1[33])
    sx_bd = _kron8(p)
    b1r = jnp.tile(b1, 8).reshape(1, 128)
    b2r = jnp.tile(b2, 8).reshape(1, 128)
    aw_bd = _kron8(aw @ jnp.ones((1, 16), jnp.float32))
    abr = jnp.full((1, 128), ab[0], jnp.float32)
    w2_bd = _kron8(w2)

    bspec = lambda nb, off: pl.BlockSpec((nb, 128), lambda i, o=off: (i + o, 0))
    wspec = lambda a: pl.BlockSpec(a.shape, lambda i: (0,) * a.ndim)
    return pl.pallas_call(
        _edge1p_body,
        grid=(E // EBLK,),
        in_specs=[bspec(EB8, 0), bspec(EB8, CBLK), bspec(EB8, 0),
                  wspec(w1a_bd), wspec(w1b_bd), wspec(sx_bd), wspec(b1r),
                  wspec(w2_bd), wspec(b2r), wspec(aw_bd), wspec(abr)],
        out_specs=bspec(EB8, 0),
        out_shape=jax.ShapeDtypeStruct((E // 8, 128), jnp.float32),
    )(gh, gh, auxp, w1a_bd, w1b_bd, sx_bd, b1r, w2_bd, b2r, aw_bd, abr)


def _edgeqp_body(hr_ref, hc_ref, aux_ref, w1a_ref, w1b_ref, sx_ref, b1_ref,
                 w2_ref, b2_ref, w3bd_ref, shq_ref, trans_ref):
    f32 = jnp.float32
    aux = aux_ref[...]
    z1 = (jnp.dot(hr_ref[...], w1a_ref[...], preferred_element_type=f32)
          + jnp.dot(hc_ref[...], w1b_ref[...], preferred_element_type=f32)
          + jnp.dot(aux, sx_ref[...], preferred_element_type=f32)
          + b1_ref[...])
    m1 = _silu(z1)
    z2 = jnp.dot(m1, w2_ref[...], preferred_element_type=f32) + b2_ref[...]
    m2 = _silu(z2)
    t_rep = jnp.dot(m2, w3bd_ref[...], preferred_element_type=f32)
    cdn = jnp.dot(aux, shq_ref[...], preferred_element_type=f32)
    trans_ref[...] = cdn * t_rep


def _edgeqp(gh, auxp, w1, b1, w2, b2, w3):
    w1a_bd = _kron8(w1[0:16])
    w1b_bd = _kron8(w1[16:32])
    p = jnp.zeros((16, 16), jnp.float32).at[0, :].set(w1[32]).at[1, :].set(w1[33])
    sx_bd = _kron8(p)
    b1r = jnp.tile(b1, 8).reshape(1, 128)
    b2r = jnp.tile(b2, 8).reshape(1, 128)
    w3_bd = _kron8(w3 @ jnp.ones((1, 16), jnp.float32))
    shq = jnp.zeros((16, 16), jnp.float32).at[jnp.arange(2, 5), jnp.arange(3)].set(1.0)
    shq_bd = _kron8(shq)
    w2_bd = _kron8(w2)

    bspec = lambda nb, off: pl.BlockSpec((nb, 128), lambda i, o=off: (i + o, 0))
    wspec = lambda a: pl.BlockSpec(a.shape, lambda i: (0,) * a.ndim)
    return pl.pallas_call(
        _edgeqp_body,
        grid=(E // EBLK,),
        in_specs=[bspec(EB8, 0), bspec(EB8, CBLK), bspec(EB8, 0),
                  wspec(w1a_bd), wspec(w1b_bd), wspec(sx_bd), wspec(b1r),
                  wspec(w2_bd), wspec(b2r), wspec(w3_bd), wspec(shq_bd)],
        out_specs=bspec(EB8, 0),
        out_shape=jax.ShapeDtypeStruct((E // 8, 128), jnp.float32),
    )(gh, gh, auxp, w1a_bd, w1b_bd, sx_bd, b1r, w2_bd, b2r, w3_bd,
      shq_bd)


def _node_body(h_ref, p_ref, w1a_ref, w1b_ref, b1_ref, w2_ref, b2_ref,
               hn_ref):
    h = h_ref[...]
    agg = (p_ref[0] + p_ref[1]) * (1.0 / NF)
    z1 = (jnp.dot(h, w1a_ref[...], preferred_element_type=jnp.float32)
          + jnp.dot(agg, w1b_ref[...], preferred_element_type=jnp.float32)
          + b1_ref[...])
    m = _silu(z1)
    hn_ref[...] = h + jnp.dot(m, w2_ref[...], preferred_element_type=jnp.float32) + b2_ref[...]


def _node(h, p, nw1, nb1, nw2, nb2):
    w1a = nw1[0:16]
    w1b = nw1[16:32]
    bspec = lambda bb, bw: pl.BlockSpec((bb, bw), lambda i: (i, 0))
    wspec = lambda a: pl.BlockSpec(a.shape, lambda i: (0,) * a.ndim)
    return pl.pallas_call(
        _node_body,
        grid=(N // NBLK,),
        in_specs=[bspec(NBLK, 16),
                  pl.BlockSpec((2, NBLK, 16), lambda i: (0, i, 0)),
                  wspec(w1a), wspec(w1b), wspec(nb1.reshape(1, 16)),
                  wspec(nw2), wspec(nb2.reshape(1, 16))],
        out_specs=bspec(NBLK, 16),
        out_shape=jax.ShapeDtypeStruct((N, 16), jnp.float32),
    )(h, p, w1a, w1b, nb1.reshape(1, 16), nw2, nb2.reshape(1, 16))


def _coord_body(x_ref, p_ref, xn_ref):
    agg = (p_ref[0] + p_ref[1]) * (1.0 / NF)
    xn_ref[...] = x_ref[...] + agg[:, 0:3]


def _coord(x, p):
    bspec = lambda bb, bw: pl.BlockSpec((bb, bw), lambda i: (i, 0))
    return pl.pallas_call(
        _coord_body,
        grid=(N // NBLK,),
        in_specs=[bspec(NBLK, 3),
                  pl.BlockSpec((2, NBLK, 16), lambda i: (0, i, 0))],
        out_specs=bspec(NBLK, 3),
        out_shape=jax.ShapeDtypeStruct((N, 3), jnp.float32),
    )(x, p)


def kernel(h, x, edge_index, node_mask, edge_mask, edge_attr,
           g0_ew1, g0_eb1, g0_ew2, g0_eb2, g0_nw1, g0_nb1, g0_nw2, g0_nb2,
           g0_aw, g0_ab, g1_ew1, g1_eb1, g1_ew2, g1_eb2, g1_nw1, g1_nb1,
           g1_nw2, g1_nb2, g1_aw, g1_ab, eq_w1, eq_b1, eq_w2, eq_b2, eq_w3):
    eflat = edge_index.reshape(2 * E)
    row = eflat[:E]
    attrp = edge_attr.reshape(E // 8, 8)
    xt = jnp.concatenate([x, jnp.zeros((N, 13), jnp.float32)], axis=1)

    gh0 = _sc_gather(h, eflat).reshape(2 * E // 8, 128)
    gx = _sc_gather(xt, eflat).reshape(2 * E // 8, 128)
    ef0p, auxp = _edge0p(gh0, gx, attrp, g0_ew1, g0_eb1, g0_ew2, g0_eb2,
                         g0_aw, g0_ab)
    p0 = _sc_scatter_partials(ef0p.reshape(E, 16), row)
    h1 = _node(h, p0, g0_nw1, g0_nb1, g0_nw2, g0_nb2)

    gh1 = _sc_gather(h1, eflat).reshape(2 * E // 8, 128)
    ef1p = _edge1p(gh1, auxp, g1_ew1, g1_eb1, g1_ew2, g1_eb2, g1_aw, g1_ab)
    p1 = _sc_scatter_partials(ef1p.reshape(E, 16), row)
    h2 = _node(h1, p1, g1_nw1, g1_nb1, g1_nw2, g1_nb2)

    gh2 = _sc_gather(h2, eflat).reshape(2 * E // 8, 128)
    transp = _edgeqp(gh2, auxp, eq_w1, eq_b1, eq_w2, eq_b2, eq_w3)
    px = _sc_scatter_partials(transp.reshape(E, 16), row)
    xn = _coord(x, px)

    return (h2, xn)


# speedup vs baseline: 14.1270x; 1.1516x over previous
import functools

import jax
import jax.numpy as jnp
from jax import lax
from jax.experimental import pallas as pl
from jax.experimental.pallas import tpu as pltpu
from jax.experimental.pallas import tpu_sc as plsc

NC = 2
NS = 16

N = 100000
E = 1600000
NF = 100.0

E2 = E // 2
EBLK = 3200
EB8 = EBLK // 8
CBLK = 250
NBLK = 4000

GCH = 1000
GRP = 5


def _silu(z):
    return z * jax.nn.sigmoid(z)


def _kron8(w):
    return jnp.kron(jnp.eye(8, dtype=jnp.float32), w)


def _sc_gather(table, idx):
    D = table.shape[1]
    M = idx.shape[0]
    per_w = M // (NC * NS)
    nsup = per_w // (GCH * GRP)

    mesh = plsc.VectorSubcoreMesh(core_axis_name="c", subcore_axis_name="s")

    @functools.partial(
        pl.kernel, mesh=mesh,
        compiler_params=pltpu.CompilerParams(use_tc_tiling_on_sc=False),
        out_type=jax.ShapeDtypeStruct((M, D), jnp.float32),
        scratch_types=[pltpu.VMEM((GRP, GCH), jnp.int32),
                       pltpu.VMEM((GRP, GCH, D), jnp.float32),
                       pltpu.SemaphoreType.DMA,
                       pltpu.SemaphoreType.DMA,
                       pltpu.SemaphoreType.DMA],
    )
    def k(table_hbm, idx_hbm, out_hbm, idx_v, rows_v, sem_i, sem_g, sem_s):
        c = lax.axis_index("c")
        s = lax.axis_index("s")
        base = (c * NS + s) * per_w

        def sup(j, carry):
            off0 = base + j * (GCH * GRP)
            cps = [pltpu.async_copy(idx_hbm.at[pl.ds(off0 + b * GCH, GCH)],
                                    idx_v.at[b], sem_i) for b in range(GRP)]
            for cp in cps:
                cp.wait()
            gps = [pltpu.async_copy(table_hbm.at[idx_v.at[b]],
                                    rows_v.at[b], sem_g) for b in range(GRP)]
            for gp in gps:
                gp.wait()
            sps = [pltpu.async_copy(rows_v.at[b],
                                    out_hbm.at[pl.ds(off0 + b * GCH, GCH)],
                                    sem_s) for b in range(GRP)]
            for sp in sps:
                sp.wait()
            return carry

        lax.fori_loop(0, nsup, sup, 0)

    return k(table, idx)


SCH = 200
SGR = 5


def _sc_scatter_partials(vals, row):
    per_core = E2 // NC
    per_w = per_core // NS
    nsup = per_w // (SCH * SGR)
    rows_per_tile = N // NS

    mesh = plsc.VectorSubcoreMesh(core_axis_name="c", subcore_axis_name="s")

    @functools.partial(
        pl.kernel, mesh=mesh,
        compiler_params=pltpu.CompilerParams(use_tc_tiling_on_sc=False),
        out_type=jax.ShapeDtypeStruct((NC, N, 16), jnp.float32),
        scratch_types=[pltpu.VMEM((SGR, SCH), jnp.int32),
                       pltpu.VMEM((SGR, SCH, 16), jnp.float32),
                       pltpu.VMEM_SHARED((N, 16), jnp.float32),
                       pltpu.SemaphoreType.DMA],
    )
    def k(vals_hbm, row_hbm, zeros_hbm, out_hbm, idx_v, val_v, table_sh, sem):
        c = lax.axis_index("c")
        s = lax.axis_index("s")
        tbase = s * rows_per_tile

        pltpu.sync_copy(zeros_hbm.at[pl.ds(tbase, rows_per_tile)],
                        table_sh.at[pl.ds(tbase, rows_per_tile)])
        plsc.subcore_barrier()

        base = c * per_core + s * per_w

        def sup(j, carry):
            off0 = base + j * (SCH * SGR)
            cps = [pltpu.async_copy(row_hbm.at[pl.ds(off0 + b * SCH, SCH)],
                                    idx_v.at[b], sem) for b in range(SGR)]
            cps += [pltpu.async_copy(vals_hbm.at[pl.ds(off0 + b * SCH, SCH)],
                                     val_v.at[b], sem) for b in range(SGR)]
            for cp in cps:
                cp.wait()
            for b in range(SGR):
                pltpu.sync_copy(val_v.at[b], table_sh.at[idx_v.at[b]],
                                add=True)
            return carry

        lax.fori_loop(0, nsup, sup, 0)
        plsc.subcore_barrier()

        pltpu.sync_copy(table_sh.at[pl.ds(tbase, rows_per_tile)],
                        out_hbm.at[c, pl.ds(tbase, rows_per_tile)])

    return k(vals, row, jnp.zeros((N, 16), jnp.float32))


def _edge0p_body(hr_ref, hc_ref, xr_ref, xc_ref, attr_ref, w1a_ref, w1b_ref,
                 sq_ref, qx_ref, shr_ref, ex_ref, w1c0_ref, w1c1_ref, b1_ref,
                 w2_ref, b2_ref, awbd_ref, ab_ref, ef_ref, aux_ref):
    f32 = jnp.float32
    cd = xr_ref[...] - xc_ref[...]
    sq = cd * cd
    radial_rep = jnp.dot(sq, sq_ref[...], preferred_element_type=f32)
    radial_x = jnp.dot(sq, qx_ref[...], preferred_element_type=f32)
    inv = 1.0 / jnp.sqrt(radial_x + 1e-8)
    cdn = cd * inv
    cdn_aux = jnp.dot(cdn, shr_ref[...], preferred_element_type=f32)
    attr_rep = jnp.dot(attr_ref[...], ex_ref[...], preferred_element_type=f32)
    lane = lax.broadcasted_iota(jnp.int32, radial_rep.shape, 1) % 16
    aux_ref[...] = (jnp.where(lane == 0, radial_rep, 0.0)
                    + jnp.where(lane == 1, attr_rep, 0.0) + cdn_aux)
    z1 = (jnp.dot(hr_ref[...], w1a_ref[...], preferred_element_type=f32)
          + jnp.dot(hc_ref[...], w1b_ref[...], preferred_element_type=f32)
          + radial_rep * w1c0_ref[...] + attr_rep * w1c1_ref[...]
          + b1_ref[...])
    m1 = _silu(z1)
    z2 = jnp.dot(m1, w2_ref[...], preferred_element_type=f32) + b2_ref[...]
    mij = _silu(z2)
    att = jax.nn.sigmoid(jnp.dot(mij, awbd_ref[...], preferred_element_type=f32)
                         + ab_ref[...])
    ef_ref[...] = mij * att


def _edge0p(gh, gx, attrp, w1, b1, w2, b2, aw, ab):
    w1a_bd = _kron8(w1[0:16])
    w1b_bd = _kron8(w1[16:32])
    q = jnp.zeros((16, 16), jnp.float32).at[0:3, :].set(1.0)
    sq_bd = _kron8(q)
    qx = jnp.zeros((16, 16), jnp.float32).at[0:3, 0:3].set(
        jnp.ones((3, 3), jnp.float32))
    qx_bd = _kron8(qx)
    shr = jnp.zeros((16, 16), jnp.float32).at[jnp.arange(3), jnp.arange(2, 5)].set(1.0)
    shr_bd = _kron8(shr)
    ex = jnp.zeros((8, 128), jnp.float32)
    ex = ex.at[jnp.repeat(jnp.arange(8), 16),
               jnp.arange(128)].set(1.0)
    w1c0 = jnp.tile(w1[32], 8).reshape(1, 128)
    w1c1 = jnp.tile(w1[33], 8).reshape(1, 128)
    b1r = jnp.tile(b1, 8).reshape(1, 128)
    b2r = jnp.tile(b2, 8).reshape(1, 128)
    aw_bd = _kron8(aw @ jnp.ones((1, 16), jnp.float32))
    abr = jnp.full((1, 128), ab[0], jnp.float32)
    w2_bd = _kron8(w2)

    bspec = lambda nb, off: pl.BlockSpec((nb, 128), lambda i, o=off: (i + o, 0))
    aspec = pl.BlockSpec((EBLK // 8, 8), lambda i: (i, 0))
    wspec = lambda a: pl.BlockSpec(a.shape, lambda i: (0,) * a.ndim)
    outs = pl.pallas_call(
        _edge0p_body,
        grid=(E2 // EBLK,),
        in_specs=[bspec(EB8, 0), bspec(EB8, CBLK), bspec(EB8, 0),
                  bspec(EB8, CBLK), aspec,
                  wspec(w1a_bd), wspec(w1b_bd), wspec(sq_bd), wspec(qx_bd),
                  wspec(shr_bd), wspec(ex), wspec(w1c0), wspec(w1c1),
                  wspec(b1r), wspec(w2_bd), wspec(b2r), wspec(aw_bd),
                  wspec(abr)],
        out_specs=[bspec(EB8, 0), bspec(EB8, 0)],
        out_shape=[jax.ShapeDtypeStruct((E2 // 8, 128), jnp.float32),
                   jax.ShapeDtypeStruct((E2 // 8, 128), jnp.float32)],
    )(gh, gh, gx, gx, attrp, w1a_bd, w1b_bd, sq_bd, qx_bd, shr_bd, ex,
      w1c0, w1c1, b1r, w2_bd, b2r, aw_bd, abr)
    return outs


def _edge1p_body(hr_ref, hc_ref, aux_ref, w1a_ref, w1b_ref, sx_ref, b1_ref,
                 w2_ref, b2_ref, awbd_ref, ab_ref, ef_ref):
    f32 = jnp.float32
    z1 = (jnp.dot(hr_ref[...], w1a_ref[...], preferred_element_type=f32)
          + jnp.dot(hc_ref[...], w1b_ref[...], preferred_element_type=f32)
          + jnp.dot(aux_ref[...], sx_ref[...], preferred_element_type=f32)
          + b1_ref[...])
    m1 = _silu(z1)
    z2 = jnp.dot(m1, w2_ref[...], preferred_element_type=f32) + b2_ref[...]
    mij = _silu(z2)
    att = jax.nn.sigmoid(jnp.dot(mij, awbd_ref[...], preferred_element_type=f32)
                         + ab_ref[...])
    ef_ref[...] = mij * att


def _edge1p(gh, auxp, w1, b1, w2, b2, aw, ab):
    w1a_bd = _kron8(w1[0:16])
    w1b_bd = _kron8(w1[16:32])
    p = jnp.zeros((16, 16), jnp.float32).at[0, :].set(w1[32]).at[1, :].set(w1[33])
    sx_bd = _kron8(p)
    b1r = jnp.tile(b1, 8).reshape(1, 128)
    b2r = jnp.tile(b2, 8).reshape(1, 128)
    aw_bd = _kron8(aw @ jnp.ones((1, 16), jnp.float32))
    abr = jnp.full((1, 128), ab[0], jnp.float32)
    w2_bd = _kron8(w2)

    bspec = lambda nb, off: pl.BlockSpec((nb, 128), lambda i, o=off: (i + o, 0))
    wspec = lambda a: pl.BlockSpec(a.shape, lambda i: (0,) * a.ndim)
    return pl.pallas_call(
        _edge1p_body,
        grid=(E2 // EBLK,),
        in_specs=[bspec(EB8, 0), bspec(EB8, CBLK), bspec(EB8, 0),
                  wspec(w1a_bd), wspec(w1b_bd), wspec(sx_bd), wspec(b1r),
                  wspec(w2_bd), wspec(b2r), wspec(aw_bd), wspec(abr)],
        out_specs=bspec(EB8, 0),
        out_shape=jax.ShapeDtypeStruct((E2 // 8, 128), jnp.float32),
    )(gh, gh, auxp, w1a_bd, w1b_bd, sx_bd, b1r, w2_bd, b2r, aw_bd, abr)


def _edgeqp_body(hr_ref, hc_ref, aux_ref, w1a_ref, w1b_ref, sx_ref, b1_ref,
                 w2_ref, b2_ref, w3bd_ref, shq_ref, trans_ref):
    f32 = jnp.float32
    aux = aux_ref[...]
    z1 = (jnp.dot(hr_ref[...], w1a_ref[...], preferred_element_type=f32)
          + jnp.dot(hc_ref[...], w1b_ref[...], preferred_element_type=f32)
          + jnp.dot(aux, sx_ref[...], preferred_element_type=f32)
          + b1_ref[...])
    m1 = _silu(z1)
    z2 = jnp.dot(m1, w2_ref[...], preferred_element_type=f32) + b2_ref[...]
    m2 = _silu(z2)
    t_rep = jnp.dot(m2, w3bd_ref[...], preferred_element_type=f32)
    cdn = jnp.dot(aux, shq_ref[...], preferred_element_type=f32)
    trans_ref[...] = cdn * t_rep


def _edgeqp(gh, auxp, w1, b1, w2, b2, w3):
    w1a_bd = _kron8(w1[0:16])
    w1b_bd = _kron8(w1[16:32])
    p = jnp.zeros((16, 16), jnp.float32).at[0, :].set(w1[32]).at[1, :].set(w1[33])
    sx_bd = _kron8(p)
    b1r = jnp.tile(b1, 8).reshape(1, 128)
    b2r = jnp.tile(b2, 8).reshape(1, 128)
    w3_bd = _kron8(w3 @ jnp.ones((1, 16), jnp.float32))
    shq = jnp.zeros((16, 16), jnp.float32).at[jnp.arange(2, 5), jnp.arange(3)].set(1.0)
    shq_bd = _kron8(shq)
    w2_bd = _kron8(w2)

    bspec = lambda nb, off: pl.BlockSpec((nb, 128), lambda i, o=off: (i + o, 0))
    wspec = lambda a: pl.BlockSpec(a.shape, lambda i: (0,) * a.ndim)
    return pl.pallas_call(
        _edgeqp_body,
        grid=(E2 // EBLK,),
        in_specs=[bspec(EB8, 0), bspec(EB8, CBLK), bspec(EB8, 0),
                  wspec(w1a_bd), wspec(w1b_bd), wspec(sx_bd), wspec(b1r),
                  wspec(w2_bd), wspec(b2r), wspec(w3_bd), wspec(shq_bd)],
        out_specs=bspec(EB8, 0),
        out_shape=jax.ShapeDtypeStruct((E2 // 8, 128), jnp.float32),
    )(gh, gh, auxp, w1a_bd, w1b_bd, sx_bd, b1r, w2_bd, b2r, w3_bd,
      shq_bd)


def _node_body(h_ref, pa_ref, pb_ref, w1a_ref, w1b_ref, b1_ref, w2_ref,
               b2_ref, hn_ref):
    h = h_ref[...]
    agg = (pa_ref[0] + pa_ref[1] + pb_ref[0] + pb_ref[1]) * (1.0 / NF)
    z1 = (jnp.dot(h, w1a_ref[...], preferred_element_type=jnp.float32)
          + jnp.dot(agg, w1b_ref[...], preferred_element_type=jnp.float32)
          + b1_ref[...])
    m = _silu(z1)
    hn_ref[...] = h + jnp.dot(m, w2_ref[...], preferred_element_type=jnp.float32) + b2_ref[...]


def _node(h, pa, pb, nw1, nb1, nw2, nb2):
    w1a = nw1[0:16]
    w1b = nw1[16:32]
    bspec = lambda bb, bw: pl.BlockSpec((bb, bw), lambda i: (i, 0))
    pspec = pl.BlockSpec((2, NBLK, 16), lambda i: (0, i, 0))
    wspec = lambda a: pl.BlockSpec(a.shape, lambda i: (0,) * a.ndim)
    return pl.pallas_call(
        _node_body,
        grid=(N // NBLK,),
        in_specs=[bspec(NBLK, 16), pspec, pspec,
                  wspec(w1a), wspec(w1b), wspec(nb1.reshape(1, 16)),
                  wspec(nw2), wspec(nb2.reshape(1, 16))],
        out_specs=bspec(NBLK, 16),
        out_shape=jax.ShapeDtypeStruct((N, 16), jnp.float32),
    )(h, pa, pb, w1a, w1b, nb1.reshape(1, 16), nw2, nb2.reshape(1, 16))


def _coord_body(x_ref, pa_ref, pb_ref, xn_ref):
    agg = (pa_ref[0] + pa_ref[1] + pb_ref[0] + pb_ref[1]) * (1.0 / NF)
    xn_ref[...] = x_ref[...] + agg[:, 0:3]


def _coord(x, pa, pb):
    bspec = lambda bb, bw: pl.BlockSpec((bb, bw), lambda i: (i, 0))
    pspec = pl.BlockSpec((2, NBLK, 16), lambda i: (0, i, 0))
    return pl.pallas_call(
        _coord_body,
        grid=(N // NBLK,),
        in_specs=[bspec(NBLK, 3), pspec, pspec],
        out_specs=bspec(NBLK, 3),
        out_shape=jax.ShapeDtypeStruct((N, 3), jnp.float32),
    )(x, pa, pb)


def kernel(h, x, edge_index, node_mask, edge_mask, edge_attr,
           g0_ew1, g0_eb1, g0_ew2, g0_eb2, g0_nw1, g0_nb1, g0_nw2, g0_nb2,
           g0_aw, g0_ab, g1_ew1, g1_eb1, g1_ew2, g1_eb2, g1_nw1, g1_nb1,
           g1_nw2, g1_nb2, g1_aw, g1_ab, eq_w1, eq_b1, eq_w2, eq_b2, eq_w3):
    eflat = edge_index.reshape(2 * E)
    rowA = eflat[0:E2]
    rowB = eflat[E2:E]
    idxA = jnp.concatenate([rowA, eflat[E:E + E2]])
    idxB = jnp.concatenate([rowB, eflat[E + E2:]])
    attrpA = edge_attr[0:E2].reshape(E2 // 8, 8)
    attrpB = edge_attr[E2:].reshape(E2 // 8, 8)
    xt = jnp.concatenate([x, jnp.zeros((N, 13), jnp.float32)], axis=1)

    rp = lambda g: g.reshape(2 * E2 // 8, 128)

    ghA = rp(_sc_gather(h, idxA))
    gxA = rp(_sc_gather(xt, idxA))
    ghB = rp(_sc_gather(h, idxB))
    gxB = rp(_sc_gather(xt, idxB))
    efA, auxA = _edge0p(ghA, gxA, attrpA, g0_ew1, g0_eb1, g0_ew2, g0_eb2,
                        g0_aw, g0_ab)
    efB, auxB = _edge0p(ghB, gxB, attrpB, g0_ew1, g0_eb1, g0_ew2, g0_eb2,
                        g0_aw, g0_ab)
    pA = _sc_scatter_partials(efA.reshape(E2, 16), rowA)
    pB = _sc_scatter_partials(efB.reshape(E2, 16), rowB)
    h1 = _node(h, pA, pB, g0_nw1, g0_nb1, g0_nw2, g0_nb2)

    ghA = rp(_sc_gather(h1, idxA))
    ghB = rp(_sc_gather(h1, idxB))
    efA = _edge1p(ghA, auxA, g1_ew1, g1_eb1, g1_ew2, g1_eb2, g1_aw, g1_ab)
    efB = _edge1p(ghB, auxB, g1_ew1, g1_eb1, g1_ew2, g1_eb2, g1_aw, g1_ab)
    pA = _sc_scatter_partials(efA.reshape(E2, 16), rowA)
    pB = _sc_scatter_partials(efB.reshape(E2, 16), rowB)
    h2 = _node(h1, pA, pB, g1_nw1, g1_nb1, g1_nw2, g1_nb2)

    ghA = rp(_sc_gather(h2, idxA))
    ghB = rp(_sc_gather(h2, idxB))
    trA = _edgeqp(ghA, auxA, eq_w1, eq_b1, eq_w2, eq_b2, eq_w3)
    trB = _edgeqp(ghB, auxB, eq_w1, eq_b1, eq_w2, eq_b2, eq_w3)
    pA = _sc_scatter_partials(trA.reshape(E2, 16), rowA)
    pB = _sc_scatter_partials(trB.reshape(E2, 16), rowB)
    xn = _coord(x, pA, pB)

    return (h2, xn)
